# baseline (speedup 1.0000x reference)
import functools

import jax
import jax.numpy as jnp
from jax import lax
from jax.experimental import pallas as pl
from jax.experimental.pallas import tpu as pltpu
from jax.experimental.pallas import tpu_sc as plsc

NC = 2
NS = 16
NW = NC * NS

B = 4096
L = 200
D = 64
HALF = 100
RPW = B // NW
CPW = RPW * (L // HALF)
NBUF = 4


def _pool_body(text_ref, emb_ref, out_ref, idx_v, bufs, out_v, s0, s1, s2, s3):
    sems = (s0, s1, s2, s3)
    wid = lax.axis_index("s") * NC + lax.axis_index("c")

    pltpu.sync_copy(text_ref.at[wid], idx_v)

    def gather(c, b):
        return pltpu.make_async_copy(
            emb_ref.at[idx_v.at[c]], bufs.at[b], sems[b])

    for b in range(NBUF):
        gather(b, b).start()

    def reduce_chunk(b):
        buf = bufs.at[b]

        def body(jj, carry):
            a0, a1, a2, a3 = carry
            for u in range(4):
                j = jj * 4 + u
                a0 = a0 + buf[j, pl.ds(0, 16)]
                a1 = a1 + buf[j, pl.ds(16, 16)]
                a2 = a2 + buf[j, pl.ds(32, 16)]
                a3 = a3 + buf[j, pl.ds(48, 16)]
            return a0, a1, a2, a3

        z = jnp.zeros((16,), jnp.float32)
        return lax.fori_loop(0, HALF // 4, body, (z, z, z, z))

    def outer(k, carry):
        for b in range(NBUF):
            c = k * NBUF + b
            gather(c, b).wait()
            a = reduce_chunk(b)
            r = k * (NBUF // 2) + (b // 2)
            if b % 2 == 0:
                for t in range(4):
                    out_v[r, pl.ds(16 * t, 16)] = a[t]
            else:
                for t in range(4):
                    out_v[r, pl.ds(16 * t, 16)] = (
                        out_v[r, pl.ds(16 * t, 16)] + a[t])

            @pl.when(k < CPW // NBUF - 1)
            def _():
                gather(c + NBUF, b).start()

        return carry

    lax.fori_loop(0, CPW // NBUF, outer, 0)
    pltpu.sync_copy(out_v, out_ref.at[pl.ds(wid * RPW, RPW)])


_pool = functools.partial(
    pl.kernel,
    out_type=jax.ShapeDtypeStruct((B, D), jnp.float32),
    mesh=plsc.VectorSubcoreMesh(
        core_axis_name="c", subcore_axis_name="s",
        num_cores=NC, num_subcores=NS),
    scratch_types=[
        pltpu.VMEM((CPW, HALF), jnp.int32),
        pltpu.VMEM((NBUF, HALF, D), jnp.float32),
        pltpu.VMEM((RPW, D), jnp.float32),
        pltpu.SemaphoreType.DMA,
        pltpu.SemaphoreType.DMA,
        pltpu.SemaphoreType.DMA,
        pltpu.SemaphoreType.DMA,
    ],
    compiler_params=pltpu.CompilerParams(use_tc_tiling_on_sc=False),
)(_pool_body)


def _mlp_body(x_ref, w1_ref, b1_ref, w2_ref, b2_ref, o_ref):
    x = x_ref[...] * (1.0 / L)
    h = lax.dot_general(x, w1_ref[...], (((1,), (1,)), ((), ())),
                        preferred_element_type=jnp.float32)
    h = jnp.maximum(h + b1_ref[...], 0.0)
    o = lax.dot_general(h, w2_ref[...], (((1,), (1,)), ((), ())),
                        preferred_element_type=jnp.float32)
    o_ref[...] = o + b2_ref[...]


def _mlp(x, w1, b1, w2, b2):
    return pl.pallas_call(
        _mlp_body,
        out_shape=jax.ShapeDtypeStruct((B, 10), jnp.float32),
    )(x, w1, b1.reshape(1, -1), w2, b2.reshape(1, -1))


def kernel(text, emb, W1, b1, W2, b2):
    text_r = text.astype(jnp.int32).reshape(NW, CPW, HALF)
    pooled_sum = _pool(text_r, emb)
    return _mlp(pooled_sum, W1, b1, W2, b2)
